# baseline (speedup 1.0000x reference)
import jax
import jax.numpy as jnp
from jax.experimental import pallas as pl
from jax.experimental.pallas import tpu as pltpu

_K = 512
_BETA = 1.0


def _kwinner_kernel(x_ref, dc_ref, out_ref):
    x = x_ref[...]
    dc = dc_ref[...]
    bsz, units = x.shape
    target = jnp.float32(_K / units)
    boost = jnp.exp(_BETA * (target - dc))
    boosted = x * boost

    bits = jax.lax.bitcast_convert_type(boosted, jnp.int32)
    w = bits ^ ((bits >> 31) & jnp.int32(0x7FFFFFFF))

    k = jnp.int32(_K)
    nchunk = 16
    csz = units // nchunk

    def count16(d16, c16):
        flags = jnp.where(d16 >= c16, jnp.int16(1), jnp.int16(0))
        s1 = flags[:, 0:csz]
        for j in range(1, nchunk):
            s1 = s1 + flags[:, j * csz:(j + 1) * csz]
        return jnp.sum(s1.astype(jnp.int32), axis=1, keepdims=True)

    d16a = (w >> 16).astype(jnp.int16)

    cnt_pos = count16(d16a, jnp.zeros((bsz, 1), jnp.int16))
    t = jnp.where(cnt_pos >= k, jnp.int32(0), jnp.int32(-2147483648))

    def step_a(i, t):
        b = 30 - i
        cand = t | (jnp.int32(1) << b)
        c16 = (cand >> 16).astype(jnp.int16)
        cnt = count16(d16a, c16)
        return jnp.where(cnt >= k, cand, t)

    t = jax.lax.fori_loop(0, 15, step_a, t)

    b0 = t
    wc = jnp.minimum(jnp.maximum(w, b0), b0 + 65535)
    d16b = (wc - b0 - 32768).astype(jnp.int16)

    def step_b(i, t):
        b = 15 - i
        cand = t | (jnp.int32(1) << b)
        c16 = (cand - b0 - 32768).astype(jnp.int16)
        flags = jnp.where(d16b >= c16, jnp.int16(1), jnp.int16(0))
        s1 = flags[:, 0:csz]
        for j in range(1, nchunk):
            s1 = s1 + flags[:, j * csz:(j + 1) * csz]
        cnt = jnp.sum(s1.astype(jnp.int32), axis=1, keepdims=True)
        return jnp.where(cnt >= k, cand, t)

    t = jax.lax.fori_loop(0, 16, step_b, t)

    out_ref[...] = jnp.where(w >= t, x, jnp.float32(0.0))


def kernel(inputs, duty_cycle):
    b, f = inputs.shape
    dc2 = duty_cycle.reshape(1, f)
    return pl.pallas_call(
        _kwinner_kernel,
        out_shape=jax.ShapeDtypeStruct((b, f), jnp.float32),
    )(inputs, dc2)


# speedup vs baseline: 1.1797x; 1.1797x over previous
import jax
import jax.numpy as jnp
from jax.experimental import pallas as pl
from jax.experimental.pallas import tpu as pltpu

_K = 512
_BETA = 1.0


def _kwinner_kernel(x_ref, dc_ref, out_ref):
    x = x_ref[...]
    dc = dc_ref[...]
    bsz, units = x.shape
    target = jnp.float32(_K / units)
    boost = jnp.exp(_BETA * (target - dc))
    boosted = x * boost

    bits = jax.lax.bitcast_convert_type(boosted, jnp.int32)
    w = bits ^ ((bits >> 31) & jnp.int32(0x7FFFFFFF))

    k = jnp.int32(_K)
    nchunk = 16
    csz = units // nchunk

    def count16(d16, c16):
        flags = jnp.where(d16 >= c16, jnp.int16(1), jnp.int16(0))
        s1 = flags[:, 0:csz]
        for j in range(1, nchunk):
            s1 = s1 + flags[:, j * csz:(j + 1) * csz]
        return jnp.sum(s1.astype(jnp.int32), axis=1, keepdims=True)

    d16a = (w >> 16).astype(jnp.int16)

    cnt_pos = count16(d16a, jnp.zeros((bsz, 1), jnp.int16))
    t = jnp.where(cnt_pos >= k, jnp.int32(0), jnp.int32(-2147483648))

    def step_a(i, t):
        b = 30 - i
        cand = t | (jnp.int32(1) << b)
        c16 = (cand >> 16).astype(jnp.int16)
        cnt = count16(d16a, c16)
        return jnp.where(cnt >= k, cand, t)

    t = jax.lax.fori_loop(0, 15, step_a, t)

    b0 = t >> 1
    hs = w >> 1
    hc = jnp.minimum(jnp.maximum(hs, b0 - 1), b0 + 32767)
    d16b = (hc - b0).astype(jnp.int16)

    def step_b(i, t):
        b = 15 - i
        cand = t | (jnp.int32(1) << b)
        c16 = ((cand >> 1) - b0).astype(jnp.int16)
        cnt = count16(d16b, c16)
        return jnp.where(cnt >= k, cand, t)

    t = jax.lax.fori_loop(0, 15, step_b, t)

    cand = t | jnp.int32(1)
    flags = jnp.where(w >= cand, jnp.int32(1), jnp.int32(0))
    cnt = jnp.sum(flags, axis=1, keepdims=True)
    t = jnp.where(cnt >= k, cand, t)

    out_ref[...] = jnp.where(w >= t, x, jnp.float32(0.0))


def kernel(inputs, duty_cycle):
    b, f = inputs.shape
    dc2 = duty_cycle.reshape(1, f)
    return pl.pallas_call(
        _kwinner_kernel,
        out_shape=jax.ShapeDtypeStruct((b, f), jnp.float32),
    )(inputs, dc2)
